# baseline (speedup 1.0000x reference)
import jax
import jax.numpy as jnp
from jax import lax
from jax.experimental import pallas as pl
from jax.experimental.pallas import tpu as pltpu
from jax.experimental.pallas import tpu_sc as plsc

N = 10000
C = 128
K = 27
EPS = 1e-5
JB = 128
NPAD = 10240
NBLK = NPAD // JB
NW = 32
TPAD = N + 16
ZROW = N
NBUF = 4


def _sc_gather_body(feats_hbm, nbr_hbm, g_hbm, idx_v, gbuf, *sems):
    sems_g = sems[:NBUF]
    sems_w = sems[NBUF:]
    wid = lax.axis_index("c") * 16 + lax.axis_index("s")

    for t in range(3):
        blk = wid + NW * t

        @pl.when(blk < NBLK)
        def _():
            base = blk * JB
            pltpu.sync_copy(nbr_hbm.at[:, pl.ds(base, JB)], idx_v)

            @pl.loop(0, K)
            def _(k):
                @pl.loop(0, JB, step=16)
                def _(c):
                    v = idx_v[k, pl.ds(c, 16)]
                    idx_v[k, pl.ds(c, 16)] = jnp.where(v >= 0, v, ZROW)

            def start_g(k):
                return pltpu.async_copy(
                    feats_hbm.at[idx_v.at[k]], gbuf.at[k % NBUF],
                    sems_g[k % NBUF])

            def start_w(k):
                return pltpu.async_copy(
                    gbuf.at[k % NBUF], g_hbm.at[k, pl.ds(base, JB)],
                    sems_w[k % NBUF])

            cps_g, cps_w = {}, {}
            for step in range(K + 2):
                if step < K:
                    if step >= NBUF:
                        cps_w[step - NBUF].wait()
                    cps_g[step] = start_g(step)
                kw = step - 2
                if 0 <= kw < K:
                    cps_g[kw].wait()
                    cps_w[kw] = start_w(kw)
            for k in range(K - NBUF, K):
                cps_w[k].wait()


def _sc_gather(feats_bf, nbr_pad):
    mesh = plsc.VectorSubcoreMesh(core_axis_name="c", subcore_axis_name="s")
    f = pl.kernel(
        _sc_gather_body,
        out_type=jax.ShapeDtypeStruct((K, NPAD, C), jnp.float32),
        mesh=mesh,
        scratch_types=[
            pltpu.VMEM((K, JB), jnp.int32),
            pltpu.VMEM((NBUF, JB, C), jnp.float32),
        ] + [pltpu.SemaphoreType.DMA] * (2 * NBUF),
    )
    return f(feats_bf, nbr_pad)


def _tc_body(g_ref, w_ref, gamma_ref, beta_ref, o_ref, acc_ref):
    k = pl.program_id(0)

    @pl.when(k == 0)
    def _():
        acc_ref[...] = jnp.zeros_like(acc_ref)

    acc_ref[...] += lax.dot_general(
        g_ref[0], w_ref[0], (((1,), (0,)), ((), ())),
        preferred_element_type=jnp.float32)

    @pl.when(k == K - 1)
    def _():
        x = acc_ref[...]
        row = lax.broadcasted_iota(jnp.int32, (NPAD, 1), 0)
        m = (row < N).astype(jnp.float32)
        xm = x * m
        mean = jnp.sum(xm, axis=0, keepdims=True) / N
        var = jnp.sum(xm * xm, axis=0, keepdims=True) / N - mean * mean
        y = (x - mean) * lax.rsqrt(var + EPS) * gamma_ref[...] + beta_ref[...]
        y = y * 0.5 * (1.0 + lax.erf(y * 0.7071067811865476))
        o_ref[...] = y[:N]


def kernel(feats, nbr_idx, W, gamma, beta):
    feats_pad = jnp.pad(feats, ((0, TPAD - N), (0, 0)))
    nbr_pad = jnp.pad(nbr_idx, ((0, 0), (0, NPAD - N)), constant_values=-1)
    g = _sc_gather(feats_pad, nbr_pad)
    out = pl.pallas_call(
        _tc_body,
        grid=(K,),
        in_specs=[
            pl.BlockSpec((1, NPAD, C), lambda k: (k, 0, 0)),
            pl.BlockSpec((1, C, C), lambda k: (k, 0, 0)),
            pl.BlockSpec((1, C), lambda k: (0, 0)),
            pl.BlockSpec((1, C), lambda k: (0, 0)),
        ],
        out_specs=pl.BlockSpec((N, C), lambda k: (0, 0)),
        out_shape=jax.ShapeDtypeStruct((N, C), jnp.float32),
        scratch_shapes=[pltpu.VMEM((NPAD, C), jnp.float32)],
    )(g, W, gamma.reshape(1, C), beta.reshape(1, C))
    return out


# speedup vs baseline: 1.0023x; 1.0023x over previous
import jax
import jax.numpy as jnp
from jax import lax
from jax.experimental import pallas as pl
from jax.experimental.pallas import tpu as pltpu
from jax.experimental.pallas import tpu_sc as plsc

N = 10000
C = 128
K = 27
EPS = 1e-5
JB = 128
NPAD = 10240
NBLK = NPAD // JB
NW = 32
TPAD = N + 16
ZROW = N
NBUF = 4


def _sc_gather_body(feats_hbm, nbr_hbm, g_hbm):
    def body(i_vmem, o_vmem):
        @pl.loop(0, JB, step=16)
        def _(c):
            v = i_vmem[0, pl.ds(c, 16)]
            i_vmem[0, pl.ds(c, 16)] = jnp.where(v >= 0, v, ZROW)

        pltpu.sync_copy(feats_hbm.at[i_vmem.at[0]], o_vmem.at[0])

    pltpu.emit_pipeline(
        body,
        grid=(K, NBLK),
        in_specs=[pl.BlockSpec((1, JB), lambda k, b: (k, b))],
        out_specs=[pl.BlockSpec((1, JB, C), lambda k, b: (k, b, 0))],
        core_axis_name=("c", "s"),
        dimension_semantics=(pltpu.PARALLEL, pltpu.PARALLEL),
    )(nbr_hbm, g_hbm)


def _sc_gather(feats_pad, nbr_pad):
    mesh = plsc.VectorSubcoreMesh(core_axis_name="c", subcore_axis_name="s")
    f = pl.kernel(
        _sc_gather_body,
        out_type=jax.ShapeDtypeStruct((K, NPAD, C), jnp.float32),
        mesh=mesh,
    )
    return f(feats_pad, nbr_pad)


def _tc_body(g_ref, w_ref, gamma_ref, beta_ref, o_ref, acc_ref):
    k = pl.program_id(0)

    @pl.when(k == 0)
    def _():
        acc_ref[...] = jnp.zeros_like(acc_ref)

    acc_ref[...] += lax.dot_general(
        g_ref[0], w_ref[0], (((1,), (0,)), ((), ())),
        preferred_element_type=jnp.float32)

    @pl.when(k == K - 1)
    def _():
        x = acc_ref[...]
        row = lax.broadcasted_iota(jnp.int32, (NPAD, 1), 0)
        m = (row < N).astype(jnp.float32)
        xm = x * m
        mean = jnp.sum(xm, axis=0, keepdims=True) / N
        var = jnp.sum(xm * xm, axis=0, keepdims=True) / N - mean * mean
        y = (x - mean) * lax.rsqrt(var + EPS) * gamma_ref[...] + beta_ref[...]
        y = y * 0.5 * (1.0 + lax.erf(y * 0.7071067811865476))
        o_ref[...] = y[:N]


def kernel(feats, nbr_idx, W, gamma, beta):
    feats_pad = jnp.pad(feats, ((0, TPAD - N), (0, 0)))
    nbr_pad = jnp.pad(nbr_idx, ((0, 0), (0, NPAD - N)), constant_values=-1)
    g = _sc_gather(feats_pad, nbr_pad)
    out = pl.pallas_call(
        _tc_body,
        grid=(K,),
        in_specs=[
            pl.BlockSpec((1, NPAD, C), lambda k: (k, 0, 0)),
            pl.BlockSpec((1, C, C), lambda k: (k, 0, 0)),
            pl.BlockSpec((1, C), lambda k: (0, 0)),
            pl.BlockSpec((1, C), lambda k: (0, 0)),
        ],
        out_specs=pl.BlockSpec((N, C), lambda k: (0, 0)),
        out_shape=jax.ShapeDtypeStruct((N, C), jnp.float32),
        scratch_shapes=[pltpu.VMEM((NPAD, C), jnp.float32)],
    )(g, W, gamma.reshape(1, C), beta.reshape(1, C))
    return out


# speedup vs baseline: 1.0037x; 1.0014x over previous
import jax
import jax.numpy as jnp
from jax import lax
from jax.experimental import pallas as pl
from jax.experimental.pallas import tpu as pltpu
from jax.experimental.pallas import tpu_sc as plsc

N = 10000
C = 128
K = 27
EPS = 1e-5
JB = 128
NPAD = 10240
NBLK = NPAD // JB
NW = 32
TPAD = N + 16
ZROW = N
NBUF = 4


def _sc_gather_body(feats_hbm, nbr_hbm, g_hbm):
    def body(i_vmem, o_vmem):
        @pl.loop(0, JB, step=16)
        def _(c):
            v = i_vmem[0, pl.ds(c, 16)]
            i_vmem[0, pl.ds(c, 16)] = jnp.where(v >= 0, v, ZROW)

        pltpu.sync_copy(feats_hbm.at[i_vmem.at[0]], o_vmem.at[0])

    pltpu.emit_pipeline(
        body,
        grid=(K, NBLK),
        in_specs=[pl.BlockSpec((1, JB), lambda k, b: (k, b))],
        out_specs=[pl.BlockSpec((1, JB, C), lambda k, b: (k, b, 0))],
        core_axis_name=("c", "s"),
        dimension_semantics=(pltpu.PARALLEL, pltpu.PARALLEL),
    )(nbr_hbm, g_hbm)


def _sc_gather(feats_pad, nbr_pad):
    mesh = plsc.VectorSubcoreMesh(core_axis_name="c", subcore_axis_name="s")
    f = pl.kernel(
        _sc_gather_body,
        out_type=jax.ShapeDtypeStruct((K, NPAD, C), jnp.float32),
        mesh=mesh,
        compiler_params=pltpu.CompilerParams(use_tc_tiling_on_sc=False),
    )
    return f(feats_pad, nbr_pad)


def _tc_body(g_ref, w_ref, gamma_ref, beta_ref, o_ref, acc_ref):
    k = pl.program_id(0)

    @pl.when(k == 0)
    def _():
        acc_ref[...] = jnp.zeros_like(acc_ref)

    acc_ref[...] += lax.dot_general(
        g_ref[0], w_ref[0], (((1,), (0,)), ((), ())),
        preferred_element_type=jnp.float32)

    @pl.when(k == K - 1)
    def _():
        x = acc_ref[...]
        row = lax.broadcasted_iota(jnp.int32, (NPAD, 1), 0)
        m = (row < N).astype(jnp.float32)
        xm = x * m
        mean = jnp.sum(xm, axis=0, keepdims=True) / N
        var = jnp.sum(xm * xm, axis=0, keepdims=True) / N - mean * mean
        y = (x - mean) * lax.rsqrt(var + EPS) * gamma_ref[...] + beta_ref[...]
        y = y * 0.5 * (1.0 + lax.erf(y * 0.7071067811865476))
        o_ref[...] = y[:N]


def kernel(feats, nbr_idx, W, gamma, beta):
    feats_pad = jnp.pad(feats, ((0, TPAD - N), (0, 0)))
    nbr_pad = jnp.pad(nbr_idx, ((0, 0), (0, NPAD - N)), constant_values=-1)
    g = _sc_gather(feats_pad, nbr_pad)
    out = pl.pallas_call(
        _tc_body,
        grid=(K,),
        in_specs=[
            pl.BlockSpec((1, NPAD, C), lambda k: (k, 0, 0)),
            pl.BlockSpec((1, C, C), lambda k: (k, 0, 0)),
            pl.BlockSpec((1, C), lambda k: (0, 0)),
            pl.BlockSpec((1, C), lambda k: (0, 0)),
        ],
        out_specs=pl.BlockSpec((N, C), lambda k: (0, 0)),
        out_shape=jax.ShapeDtypeStruct((N, C), jnp.float32),
        scratch_shapes=[pltpu.VMEM((NPAD, C), jnp.float32)],
    )(g, W, gamma.reshape(1, C), beta.reshape(1, C))
    return out
